# baseline (speedup 1.0000x reference)
import functools

import jax
import jax.numpy as jnp
from jax import lax
from jax.experimental import pallas as pl
from jax.experimental.pallas import tpu as pltpu
from jax.experimental.pallas import tpu_sc as plsc

_VOCAB = 100000
_MAX_LEN = 2048
_DIM = 1024
_BATCH = 4

_NC = 2
_NS = 16
_NW = _NC * _NS
_T_PER_W = _MAX_LEN // _NW
_CHUNK = 16
_NCHUNK = _BATCH * _T_PER_W // _CHUNK
_NH = _T_PER_W // _CHUNK
_LANES = 16
_NBUF = 5
_GDEPTH = 3

_mesh = plsc.VectorSubcoreMesh(core_axis_name="c", subcore_axis_name="s")


@functools.partial(
    pl.kernel,
    mesh=_mesh,
    out_type=jax.ShapeDtypeStruct((_BATCH * _MAX_LEN, _DIM), jnp.float32),
    scratch_types=[
        pltpu.VMEM((_BATCH * _T_PER_W,), jnp.int32),
        pltpu.VMEM((_NBUF * _CHUNK, _DIM), jnp.float32),
        pltpu.VMEM((2 * _CHUNK, _DIM), jnp.float32),
        pltpu.SemaphoreType.DMA,
        pltpu.SemaphoreType.DMA,
        pltpu.SemaphoreType.DMA,
        pltpu.SemaphoreType.DMA,
    ],
)
def _embed(idx_hbm, table_hbm, pos_hbm, out_hbm,
           idx_v, big, pos_big, isem, psem, gsem, wsem):
    wid = lax.axis_index("s") * _NC + lax.axis_index("c")
    t0 = wid * _T_PER_W

    for b in range(_BATCH):
        pltpu.async_copy(idx_hbm.at[b, pl.ds(t0, _T_PER_W)],
                         idx_v.at[pl.ds(b * _T_PER_W, _T_PER_W)], isem)

    def pos_load(h, hslot):
        return pltpu.async_copy(
            pos_hbm.at[pl.ds(t0 + h * _CHUNK, _CHUNK)],
            pos_big.at[pl.ds(hslot * _CHUNK, _CHUNK)], psem)

    def gather(j, slot):
        b = lax.rem(j, _BATCH)
        h = lax.div(j, _BATCH)
        ioff = pl.multiple_of(b * _T_PER_W + h * _CHUNK, _CHUNK)
        return pltpu.async_copy(
            table_hbm.at[idx_v.at[pl.ds(ioff, _CHUNK)]],
            big.at[pl.ds(pl.multiple_of(slot * _CHUNK, _CHUNK), _CHUNK)],
            gsem)

    pos_load(0, 0)
    pos_load(1, 1)
    for hnd in range(_BATCH):
        pltpu.make_async_copy(idx_hbm.at[0, pl.ds(t0, _T_PER_W)],
                              idx_v.at[pl.ds(0, _T_PER_W)], isem).wait()
    for j in range(_GDEPTH):
        gather(j, j)

    def chunk_wait_g():
        pltpu.make_async_copy(table_hbm.at[pl.ds(0, _CHUNK)],
                              big.at[pl.ds(0, _CHUNK)], gsem).wait()

    def chunk_wait_p():
        pltpu.make_async_copy(pos_hbm.at[pl.ds(0, _CHUNK)],
                              pos_big.at[pl.ds(0, _CHUNK)], psem).wait()

    def chunk_wait_w():
        pltpu.make_async_copy(big.at[pl.ds(0, _CHUNK)],
                              out_hbm.at[pl.ds(0, _CHUNK)], wsem).wait()

    def body(c, _):
        slot = lax.rem(c, _NBUF)
        b = lax.rem(c, _BATCH)
        h = lax.div(c, _BATCH)
        hslot = lax.rem(h, 2)
        chunk_wait_g()

        @pl.when(b == 0)
        def _():
            chunk_wait_p()

        row0 = pl.multiple_of(slot * _CHUNK, _CHUNK)
        prow0 = pl.multiple_of(hslot * _CHUNK, _CHUNK)

        def add_row(r, _):
            for cc in range(_DIM // _LANES):
                sl = pl.ds(cc * _LANES, _LANES)
                plsc.addupdate(big.at[row0 + r, sl], pos_big[prow0 + r, sl])
            return 0

        lax.fori_loop(0, _CHUNK, add_row, 0)

        @pl.when((b == _BATCH - 1) & (h + 2 <= _NH - 1))
        def _():
            pos_load(h + 2, hslot)

        orow = pl.multiple_of(b * _MAX_LEN + t0 + h * _CHUNK, _CHUNK)
        pltpu.async_copy(big.at[pl.ds(row0, _CHUNK)],
                         out_hbm.at[pl.ds(orow, _CHUNK)], wsem)

        @pl.when((c >= _NBUF - _GDEPTH) & (c + _GDEPTH < _NCHUNK))
        def _():
            chunk_wait_w()

        @pl.when(c + _GDEPTH < _NCHUNK)
        def _():
            gather(c + _GDEPTH, lax.rem(c + _GDEPTH, _NBUF))

        return 0

    lax.fori_loop(0, _NCHUNK, body, 0)
    for _ in range(_NBUF):
        chunk_wait_w()


def kernel(inputs, token_table, pos_table):
    out = _embed(inputs.astype(jnp.int32), token_table, pos_table)
    return out.reshape(_BATCH, _MAX_LEN, _DIM)
